# baseline (speedup 1.0000x reference)
import functools

import jax
import jax.numpy as jnp
from jax import lax
from jax.experimental import pallas as pl
from jax.experimental.pallas import tpu as pltpu
from jax.experimental.pallas import tpu_sc as plsc

B = 4
S = 4096
D_MODEL = 4096
H = 32
D_HEAD = D_MODEL // H
SB = 512
HB = 32

NC = 2
NS = 16
NW = NC * NS
ROWS_TOTAL = B * S * H
PER_W = ROWS_TOTAL // NW
CH = 128
NCH = PER_W // CH


def _proj_kernel(x_ref, w_ref, keys_ref):
    xb = x_ref[0]
    w = w_ref[...]
    proj = lax.dot_general(w, xb, (((1,), (1,)), ((), ())),
                           preferred_element_type=jnp.float32)
    keys_ref[0] = proj[:H, :] / proj[H:, :]


def _projection(x, W):
    return pl.pallas_call(
        _proj_kernel,
        grid=(B, S // SB),
        in_specs=[
            pl.BlockSpec((1, SB, D_MODEL), lambda b, s: (b, s, 0)),
            pl.BlockSpec((2 * H, D_MODEL), lambda b, s: (0, 0)),
        ],
        out_specs=pl.BlockSpec((1, H, SB), lambda b, s: (b, 0, s)),
        out_shape=jax.ShapeDtypeStruct((B, H, S), jnp.float32),
    )(x, W)


def _sort_kernel(keys_ref, flat_ref):
    kf = keys_ref[0]
    ib = lax.bitcast_convert_type(kf, jnp.int32)
    key = ib ^ lax.shift_right_logical(
        lax.shift_right_arithmetic(ib, 31), 1)
    lane = lax.broadcasted_iota(jnp.int32, (HB, S), 1)
    v = (key & jnp.int32(~0xFFF)) | lane
    kk = 2
    while kk <= S:
        j = kk // 2
        while j >= 1:
            is_lo = (lane & j) == 0
            up = (lane & kk) == 0
            pk = jnp.where(is_lo, pltpu.roll(v, S - j, 1), pltpu.roll(v, j, 1))
            lo = jnp.where(is_lo, v, pk)
            hi = jnp.where(is_lo, pk, v)
            doswap = (up & (lo > hi)) | (~up & (lo < hi))
            v = jnp.where(doswap, pk, v)
            j //= 2
        kk *= 2
    p = v & 0xFFF
    b = pl.program_id(0)
    h = (lax.broadcasted_iota(jnp.int32, (HB, S), 0)
         + pl.program_id(1) * HB)
    flat_ref[0] = (b * (S * H) + lax.shift_right_logical(p, 3) * (H * 8)
                   + h * 8 + (p & 7))


def _argsort_flat(keys):
    return pl.pallas_call(
        _sort_kernel,
        grid=(B, H // HB),
        in_specs=[pl.BlockSpec((1, HB, S), lambda b, hb: (b, hb, 0))],
        out_specs=pl.BlockSpec((1, HB, S), lambda b, hb: (b, hb, 0)),
        out_shape=jax.ShapeDtypeStruct((B, H, S), jnp.int32),
    )(keys)


K = 2


@functools.partial(
    pl.kernel,
    out_type=jax.ShapeDtypeStruct((ROWS_TOTAL, D_HEAD), jnp.float32),
    mesh=plsc.VectorSubcoreMesh(core_axis_name="c", subcore_axis_name="s"),
    scratch_types=[
        pltpu.VMEM((NCH, CH), jnp.int32),
        pltpu.VMEM((CH, D_HEAD), jnp.float32),
        pltpu.VMEM((CH, D_HEAD), jnp.float32),
        pltpu.VMEM((CH, D_HEAD), jnp.float32),
        pltpu.VMEM((CH, D_HEAD), jnp.float32),
        pltpu.VMEM((CH, D_HEAD), jnp.float32),
        pltpu.VMEM((CH, D_HEAD), jnp.float32),
        pltpu.SemaphoreType.DMA,
        pltpu.SemaphoreType.DMA,
        pltpu.SemaphoreType.DMA,
        pltpu.SemaphoreType.DMA,
        pltpu.SemaphoreType.DMA,
        pltpu.SemaphoreType.DMA,
    ],
)
def _sc_permute_copy(x_hbm, idx_hbm, out_hbm,
                     idx_all, r0, r1, r2, r3, r4, r5,
                     sem_ga, sem_gb, sem_gc, sem_sa, sem_sb, sem_sc):
    wid = lax.axis_index("s") * NC + lax.axis_index("c")
    sets = (((r0, r1), sem_ga, sem_sa),
            ((r2, r3), sem_gb, sem_sb),
            ((r4, r5), sem_gc, sem_sc))

    pltpu.sync_copy(idx_hbm.at[wid], idx_all)

    NSETS = len(sets)

    def run_group(j0, bufs, sem_g, sem_s, drain_prev):
        @pl.when(drain_prev)
        def _():
            for t in range(K):
                pltpu.make_async_copy(bufs[t],
                                      out_hbm.at[idx_all.at[j0 - NSETS * K
                                                            + t]],
                                      sem_s).wait()
        for t in range(K):
            pltpu.async_copy(x_hbm.at[idx_all.at[j0 + t]], bufs[t], sem_g)
        for t in range(K):
            pltpu.make_async_copy(x_hbm.at[idx_all.at[j0 + t]], bufs[t],
                                  sem_g).wait()
        for t in range(K):
            pltpu.async_copy(bufs[t], out_hbm.at[idx_all.at[j0 + t]], sem_s)

    def body(g, _):
        j0 = NSETS * K * g
        for si, (bufs, sem_g, sem_s) in enumerate(sets):
            run_group(j0 + si * K, bufs, sem_g, sem_s, g >= 1)
        return 0

    NFULL = NCH // (NSETS * K)
    lax.fori_loop(0, NFULL, body, 0)
    TAIL0 = NFULL * NSETS * K
    for j0 in range(TAIL0, NCH, K):
        run_group(j0, sets[0][0], sets[0][1], sets[0][2], TAIL0 > 0)
    for si, (bufs, _, sem_s) in enumerate(sets):
        for t in range(K):
            pltpu.make_async_copy(bufs[t],
                                  out_hbm.at[idx_all.at[si * K + t]],
                                  sem_s).wait()


def kernel(x, W):
    keys = _projection(x, W)
    flat = _argsort_flat(keys)
    idx2 = flat.reshape(NW, NCH, CH)
    xp = (x.reshape(B, S // 8, 8, H, D_HEAD)
           .transpose(0, 1, 3, 2, 4)
           .reshape(ROWS_TOTAL, D_HEAD))
    outp = _sc_permute_copy(xp, idx2)
    return (outp.reshape(B, S // 8, H, 8, D_HEAD)
                .transpose(0, 1, 3, 2, 4)
                .reshape(B, S, D_MODEL))


# speedup vs baseline: 1.0021x; 1.0021x over previous
import functools

import jax
import jax.numpy as jnp
from jax import lax
from jax.experimental import pallas as pl
from jax.experimental.pallas import tpu as pltpu
from jax.experimental.pallas import tpu_sc as plsc

B = 4
S = 4096
D_MODEL = 4096
H = 32
D_HEAD = D_MODEL // H
SB = 512
HB = 32

NC = 2
NS = 16
NW = NC * NS
ROWS_TOTAL = B * S * H
PER_W = ROWS_TOTAL // NW
CH = 128
NCH = PER_W // CH


def _proj_kernel(x_ref, w_ref, keys_ref):
    xb = x_ref[0]
    w = w_ref[...]
    proj = lax.dot_general(w, xb, (((1,), (1,)), ((), ())),
                           preferred_element_type=jnp.float32)
    keys_ref[0] = proj[:H, :] / proj[H:, :]


def _projection(x, W):
    return pl.pallas_call(
        _proj_kernel,
        grid=(B, S // SB),
        in_specs=[
            pl.BlockSpec((1, SB, D_MODEL), lambda b, s: (b, s, 0)),
            pl.BlockSpec((2 * H, D_MODEL), lambda b, s: (0, 0)),
        ],
        out_specs=pl.BlockSpec((1, H, SB), lambda b, s: (b, 0, s)),
        out_shape=jax.ShapeDtypeStruct((B, H, S), jnp.float32),
    )(x, W)


def _sort_kernel(keys_ref, flat_ref):
    kf = keys_ref[0]
    ib = lax.bitcast_convert_type(kf, jnp.int32)
    key = ib ^ lax.shift_right_logical(
        lax.shift_right_arithmetic(ib, 31), 1)
    lane = lax.broadcasted_iota(jnp.int32, (HB, S), 1)
    v = (key & jnp.int32(~0xFFF)) | lane
    kk = 2
    while kk <= S:
        j = kk // 2
        while j >= 1:
            is_lo = (lane & j) == 0
            up = (lane & kk) == 0
            pk = jnp.where(is_lo, pltpu.roll(v, S - j, 1), pltpu.roll(v, j, 1))
            lo = jnp.where(is_lo, v, pk)
            hi = jnp.where(is_lo, pk, v)
            doswap = (up & (lo > hi)) | (~up & (lo < hi))
            v = jnp.where(doswap, pk, v)
            j //= 2
        kk *= 2
    p = v & 0xFFF
    b = pl.program_id(0)
    h = (lax.broadcasted_iota(jnp.int32, (HB, S), 0)
         + pl.program_id(1) * HB)
    flat_ref[0] = (b * (S * H) + lax.shift_right_logical(p, 3) * (H * 8)
                   + h * 8 + (p & 7))


def _argsort_flat(keys):
    return pl.pallas_call(
        _sort_kernel,
        grid=(B, H // HB),
        in_specs=[pl.BlockSpec((1, HB, S), lambda b, hb: (b, hb, 0))],
        out_specs=pl.BlockSpec((1, HB, S), lambda b, hb: (b, hb, 0)),
        out_shape=jax.ShapeDtypeStruct((B, H, S), jnp.int32),
    )(keys)


K = 2


@functools.partial(
    pl.kernel,
    out_type=jax.ShapeDtypeStruct((ROWS_TOTAL, D_HEAD), jnp.float32),
    mesh=plsc.VectorSubcoreMesh(core_axis_name="c", subcore_axis_name="s"),
    scratch_types=[
        pltpu.VMEM((NCH, CH), jnp.int32),
        pltpu.VMEM((CH, D_HEAD), jnp.float32),
        pltpu.VMEM((CH, D_HEAD), jnp.float32),
        pltpu.VMEM((CH, D_HEAD), jnp.float32),
        pltpu.VMEM((CH, D_HEAD), jnp.float32),
        pltpu.SemaphoreType.DMA,
        pltpu.SemaphoreType.DMA,
        pltpu.SemaphoreType.DMA,
        pltpu.SemaphoreType.DMA,
    ],
)
def _sc_permute_copy(x_hbm, idx_hbm, out_hbm,
                     idx_all, r0, r1, r2, r3,
                     sem_ga, sem_gb, sem_sa, sem_sb):
    wid = lax.axis_index("s") * NC + lax.axis_index("c")
    sets = (((r0, r1), sem_ga, sem_sa),
            ((r2, r3), sem_gb, sem_sb))

    pltpu.sync_copy(idx_hbm.at[wid], idx_all)

    NSETS = len(sets)

    def run_group(j0, bufs, sem_g, sem_s, drain_prev):
        @pl.when(drain_prev)
        def _():
            for t in range(K):
                pltpu.make_async_copy(bufs[t],
                                      out_hbm.at[idx_all.at[j0 - NSETS * K
                                                            + t]],
                                      sem_s).wait()
        for t in range(K):
            pltpu.async_copy(x_hbm.at[idx_all.at[j0 + t]], bufs[t], sem_g)
        for t in range(K):
            pltpu.make_async_copy(x_hbm.at[idx_all.at[j0 + t]], bufs[t],
                                  sem_g).wait()
        for t in range(K):
            pltpu.async_copy(bufs[t], out_hbm.at[idx_all.at[j0 + t]], sem_s)

    def body(g, _):
        j0 = NSETS * K * g
        for si, (bufs, sem_g, sem_s) in enumerate(sets):
            run_group(j0 + si * K, bufs, sem_g, sem_s, g >= 1)
        return 0

    lax.fori_loop(0, NCH // (NSETS * K), body, 0)
    for si, (bufs, _, sem_s) in enumerate(sets):
        for t in range(K):
            pltpu.make_async_copy(bufs[t],
                                  out_hbm.at[idx_all.at[si * K + t]],
                                  sem_s).wait()


def kernel(x, W):
    keys = _projection(x, W)
    flat = _argsort_flat(keys)
    idx2 = flat.reshape(NW, NCH, CH)
    xp = (x.reshape(B, S // 8, 8, H, D_HEAD)
           .transpose(0, 1, 3, 2, 4)
           .reshape(ROWS_TOTAL, D_HEAD))
    outp = _sc_permute_copy(xp, idx2)
    return (outp.reshape(B, S // 8, H, 8, D_HEAD)
                .transpose(0, 1, 3, 2, 4)
                .reshape(B, S, D_MODEL))
